# baseline (speedup 1.0000x reference)
import functools

import jax
import jax.numpy as jnp
from jax import lax
from jax.experimental import pallas as pl
from jax.experimental.pallas import tpu as pltpu
from jax.experimental.pallas import tpu_sc as plsc

D = 64
B = 16384
V = 1000000
W_DT = 0.75
W_IDX = 0.25
MAXN = 2.0

NC = 2
NS = 16
L = 16
NW = NC * NS
BPW = B // NW
NBUF = 6
BLK = 128
VTAIL = V - 64


def _rsqrt(x):
    x = jnp.maximum(x, jnp.float32(1e-20))
    i = plsc.bitcast(x, jnp.int32)
    i = jnp.int32(0x5F3759DF) - (i >> 1)
    y = plsc.bitcast(i, jnp.float32)
    for _ in range(3):
        y = y * (jnp.float32(1.5) - jnp.float32(0.5) * x * y * y)
    return y


_mesh = plsc.VectorSubcoreMesh(core_axis_name="c", subcore_axis_name="s")


@functools.partial(
    pl.kernel,
    mesh=_mesh,
    compiler_params=pltpu.CompilerParams(needs_layout_passes=False),
    out_type=jax.ShapeDtypeStruct((B // 2, 2 * D), jnp.float32),
    scratch_types=[
        pltpu.VMEM((BPW,), jnp.int32),
        pltpu.VMEM((BPW,), jnp.int32),
        pltpu.VMEM((NBUF + 1, 8, 8, BLK), jnp.float32),
        pltpu.VMEM((NBUF + 1, 8, 8, BLK), jnp.float32),
        pltpu.VMEM((2, 8, 2 * D), jnp.float32),
        pltpu.SemaphoreType.DMA((NBUF,)),
        pltpu.SemaphoreType.DMA((NBUF,)),
        pltpu.SemaphoreType.DMA((2,)),
    ],
)
def _sc_embed(dt_idx_hbm, c_idx_hbm, t1_hbm, t2_hbm, tail1_hbm, tail2_hbm,
              out_hbm, idx1_v, idx2_v, blk1_v, blk2_v, out_v, sems1, sems2,
              osems):
    wid = lax.axis_index("s") * NC + lax.axis_index("c")
    NG = BPW // L

    pltpu.sync_copy(dt_idx_hbm.at[wid], idx1_v)
    pltpu.sync_copy(c_idx_hbm.at[wid], idx2_v)
    pltpu.sync_copy(tail1_hbm, blk1_v.at[NBUF])
    pltpu.sync_copy(tail2_hbm, blk2_v.at[NBUF])

    def fetch_row(iv1, iv2, j, slot):
        i1 = jnp.where(iv1[j] >= VTAIL, jnp.int32(0), iv1[j])
        i2 = jnp.where(iv2[j] >= VTAIL, jnp.int32(0), iv2[j])
        c1 = pl.multiple_of((i1 >> 7) << 7, BLK)
        c2 = pl.multiple_of((i2 >> 7) << 7, BLK)
        pltpu.async_copy(t1_hbm.at[:, :, pl.ds(c1, BLK)], blk1_v.at[slot],
                         sems1.at[slot])
        pltpu.async_copy(t2_hbm.at[:, :, pl.ds(c2, BLK)], blk2_v.at[slot],
                         sems2.at[slot])

    def compute_row(iv1, iv2, g, j, slot):
        pltpu.make_async_copy(t1_hbm.at[:, :, pl.ds(0, BLK)],
                              blk1_v.at[slot], sems1.at[slot]).wait()
        pltpu.make_async_copy(t2_hbm.at[:, :, pl.ds(0, BLK)],
                              blk2_v.at[slot], sems2.at[slot]).wait()
        i1, i2 = iv1[j], iv2[j]
        tl1 = i1 >= VTAIL
        tl2 = i2 >= VTAIL
        sel1 = jnp.full((L,), jnp.where(tl1, jnp.int32(NBUF), slot))
        sel2 = jnp.full((L,), jnp.where(tl2, jnp.int32(NBUF), slot))
        col1 = jnp.full((L,), (i1 & jnp.int32(BLK - 1))
                        + jnp.where(tl1, jnp.int32(64), jnp.int32(0)))
        col2 = jnp.full((L,), (i2 & jnp.int32(BLK - 1))
                        + jnp.where(tl2, jnp.int32(64), jnp.int32(0)))
        lane = lax.iota(jnp.int32, L)
        rv = lane & jnp.int32(7)
        c1 = []
        c2 = []
        for k in range(D // L):
            trv = (jnp.int32(k * L) + lane) >> 3
            c1.append(plsc.load_gather(blk1_v, [sel1, trv, rv, col1]))
            c2.append(plsc.load_gather(blk2_v, [sel2, trv, rv, col2]))
        t1 = c1[0] * c1[0]
        t2 = c2[0] * c2[0]
        for k in range(1, D // L):
            t1 = t1 + c1[k] * c1[k]
            t2 = t2 + c2[k] * c2[k]
        a1 = jnp.float32(W_DT) * jnp.minimum(
            jnp.float32(1.0),
            jnp.float32(MAXN) * _rsqrt(jnp.full((L,), jnp.sum(t1))))
        a2 = jnp.float32(W_IDX) * jnp.minimum(
            jnp.float32(1.0),
            jnp.float32(MAXN) * _rsqrt(jnp.full((L,), jnp.sum(t2))))
        par = lax.rem(g, 2)
        half = j % 2
        for k in range(D // L):
            out_v[par, j // 2,
                  pl.ds(half * D + k * L, L)] = a1 * c1[k] + a2 * c2[k]

    def body(g, _):
        iv1 = idx1_v[pl.ds(g * L, L)]
        iv2 = idx2_v[pl.ds(g * L, L)]
        gn = jnp.minimum(g + 1, jnp.int32(NG - 1))
        nv1 = idx1_v[pl.ds(gn * L, L)]
        nv2 = idx2_v[pl.ds(gn * L, L)]
        par = lax.rem(g, 2)
        r0 = g * L

        @pl.when(g >= 2)
        def _():
            pltpu.make_async_copy(out_hbm.at[pl.ds(0, 8)], out_v.at[par],
                                  osems.at[par]).wait()

        PD = NBUF - 1
        for j in range(L):
            sl = lax.rem(r0 + j + PD, jnp.int32(NBUF))
            if j + PD < L:
                fetch_row(iv1, iv2, j + PD, sl)
            else:
                fetch_row(nv1, nv2, j + PD - L, sl)
            compute_row(iv1, iv2, g, j, lax.rem(r0 + j, jnp.int32(NBUF)))

        pltpu.async_copy(out_v.at[par],
                         out_hbm.at[pl.ds(wid * (BPW // 2) + g * 8, 8)],
                         osems.at[par])
        return 0

    iv1_0 = idx1_v[pl.ds(0, L)]
    iv2_0 = idx2_v[pl.ds(0, L)]
    for j in range(NBUF - 1):
        fetch_row(iv1_0, iv2_0, j, jnp.int32(j))
    lax.fori_loop(0, NG, body, 0)
    for j in range(NBUF - 1):
        sl = lax.rem(jnp.int32(BPW + j), jnp.int32(NBUF))
        pltpu.make_async_copy(t1_hbm.at[:, :, pl.ds(0, BLK)],
                              blk1_v.at[sl], sems1.at[sl]).wait()
        pltpu.make_async_copy(t2_hbm.at[:, :, pl.ds(0, BLK)],
                              blk2_v.at[sl], sems2.at[sl]).wait()
    for p in range(2):
        pltpu.make_async_copy(out_hbm.at[pl.ds(0, 8)], out_v.at[p],
                              osems.at[p]).wait()


def kernel(dt_indices, const_indices, dt_table, const_table):
    dt_idx = dt_indices.reshape(NW, BPW)
    c_idx = const_indices.reshape(NW, BPW)
    t1 = dt_table.T.reshape(8, 8, V)
    t2 = const_table.T.reshape(8, 8, V)
    tail1 = dt_table.T[:, V - BLK:].reshape(8, 8, BLK)
    tail2 = const_table.T[:, V - BLK:].reshape(8, 8, BLK)
    out = _sc_embed(dt_idx, c_idx, t1, t2, tail1, tail2)
    return out.reshape(B, D)
